# Initial kernel scaffold; baseline (speedup 1.0000x reference)
import jax
import jax.numpy as jnp
from jax import lax
from jax.experimental import pallas as pl
from jax.experimental.pallas import tpu as pltpu
from jax.experimental.pallas import tpu_sc as plsc

NUM_ROWS = 1_000_000
DIM = 32

_info = plsc.get_sparse_core_info()
_NC = _info.num_cores
_NS = _info.num_subcores
_NW = _NC * _NS

_IW = 128
_CH = 8


def _body(table_hbm, idx_hbm, out_hbm, idx_v, rows_v, sem):
    n_rows_idx = idx_hbm.shape[0]
    rows_per_w = n_rows_idx // _NW
    n_chunks = rows_per_w // _CH
    wid = lax.axis_index("s") * _NC + lax.axis_index("c")
    w_row0 = wid * rows_per_w

    def chunk(g, carry):
        row0 = w_row0 + g * _CH
        pltpu.sync_copy(idx_hbm.at[pl.ds(row0, _CH)], idx_v)
        cps = [
            pltpu.async_copy(
                table_hbm.at[idx_v.at[j]],
                rows_v.at[pl.ds(j * _IW, _IW)],
                sem,
            )
            for j in range(_CH)
        ]
        for cp in cps:
            cp.wait()
        pltpu.sync_copy(rows_v, out_hbm.at[pl.ds(row0 * _IW, _CH * _IW)])
        return carry

    lax.fori_loop(0, n_chunks, chunk, 0)


def kernel(x, weight):
    b0, b1 = x.shape
    flat = (x.reshape(-1).astype(jnp.int32)).reshape(-1, _IW)
    call = pl.kernel(
        _body,
        out_type=jax.ShapeDtypeStruct((b0 * b1, DIM), jnp.float32),
        mesh=plsc.VectorSubcoreMesh(core_axis_name="c", subcore_axis_name="s"),
        scratch_types=[
            pltpu.VMEM((_CH, _IW), jnp.int32),
            pltpu.VMEM((_CH * _IW, DIM), jnp.float32),
            pltpu.SemaphoreType.DMA,
        ],
    )
    out = call(weight, flat)
    return out.reshape(b0, b1, DIM)


# speedup vs baseline: 1.1016x; 1.1016x over previous
import jax
import jax.numpy as jnp
from jax import lax
from jax.experimental import pallas as pl
from jax.experimental.pallas import tpu as pltpu
from jax.experimental.pallas import tpu_sc as plsc

NUM_ROWS = 1_000_000
DIM = 32

_info = plsc.get_sparse_core_info()
_NC = _info.num_cores
_NS = _info.num_subcores
_NW = _NC * _NS

_IW = 128
_CH = 8


def _body(table_hbm, idx_hbm, out_hbm, idx_v, rows_v, sem):
    n_rows_idx = idx_hbm.shape[0]
    rows_per_w = n_rows_idx // _NW
    n_chunks = rows_per_w // _CH
    wid = lax.axis_index("s") * _NC + lax.axis_index("c")
    w_row0 = wid * rows_per_w

    def chunk(g, carry):
        row0 = w_row0 + g * _CH
        pltpu.sync_copy(idx_hbm.at[pl.ds(row0, _CH)], idx_v)
        cps = [
            pltpu.async_copy(
                table_hbm.at[idx_v.at[j]],
                rows_v.at[pl.ds(j * _IW, _IW)],
                sem,
            )
            for j in range(_CH)
        ]
        for cp in cps:
            cp.wait()
        pltpu.sync_copy(rows_v, out_hbm.at[pl.ds(row0 * _IW, _CH * _IW)])
        return carry

    lax.fori_loop(0, n_chunks, chunk, 0)


def kernel(x, weight):
    b0, b1 = x.shape
    flat = (x.reshape(-1).astype(jnp.int32)).reshape(-1, _IW)
    call = pl.kernel(
        _body,
        out_type=jax.ShapeDtypeStruct((b0 * b1, DIM), jnp.float32),
        mesh=plsc.VectorSubcoreMesh(core_axis_name="c", subcore_axis_name="s"),
        scratch_types=[
            pltpu.VMEM((_CH, _IW), jnp.int32),
            pltpu.VMEM((_CH * _IW, DIM), jnp.float32),
            pltpu.SemaphoreType.DMA,
        ],
        compiler_params=pltpu.CompilerParams(use_tc_tiling_on_sc=False),
    )
    out = call(weight, flat)
    return out.reshape(b0, b1, DIM)


# speedup vs baseline: 2.5550x; 2.3193x over previous
import jax
import jax.numpy as jnp
from jax import lax
from jax.experimental import pallas as pl
from jax.experimental.pallas import tpu as pltpu
from jax.experimental.pallas import tpu_sc as plsc

NUM_ROWS = 1_000_000
DIM = 32
PADB = 104

_info = plsc.get_sparse_core_info()
_NC = _info.num_cores
_NS = _info.num_subcores
_NW = _NC * _NS

_CH = 8


def _gather_body(table_hbm, idx_hbm, out_hbm, idx_v, rows_v, sem):
    b0, b1 = idx_hbm.shape
    rows_per_w = b0 // _NW
    n_chunks = rows_per_w // _CH
    wid = lax.axis_index("s") * _NC + lax.axis_index("c")
    w_row0 = wid * rows_per_w

    def chunk(g, carry):
        i0 = w_row0 + g * _CH
        pltpu.sync_copy(idx_hbm.at[pl.ds(i0, _CH)], idx_v)
        cps = [
            pltpu.async_copy(
                table_hbm.at[idx_v.at[r]],
                rows_v.at[pl.ds(PADB * r, b1)],
                sem,
            )
            for r in range(_CH)
        ]
        for cp in cps:
            cp.wait()
        pltpu.sync_copy(
            rows_v,
            out_hbm.at[pl.ds(PADB * i0, PADB * _CH), pl.ds(0, DIM)],
        )
        return carry

    lax.fori_loop(0, n_chunks, chunk, 0)


_RB = 8


def _unpack_body(in_ref, out_ref):
    for r in range(_RB):
        out_ref[r] = in_ref[pl.ds(PADB * r, out_ref.shape[1]), pl.ds(0, DIM)]


def kernel(x, weight):
    b0, b1 = x.shape
    xi = x.astype(jnp.int32)
    gather = pl.kernel(
        _gather_body,
        out_type=jax.ShapeDtypeStruct((b0 * PADB, 128), jnp.float32),
        mesh=plsc.VectorSubcoreMesh(core_axis_name="c", subcore_axis_name="s"),
        scratch_types=[
            pltpu.VMEM((_CH, b1), jnp.int32),
            pltpu.VMEM((PADB * _CH, DIM), jnp.float32),
            pltpu.SemaphoreType.DMA,
        ],
        compiler_params=pltpu.CompilerParams(use_tc_tiling_on_sc=False),
    )
    staged = gather(weight, xi)

    out = pl.pallas_call(
        _unpack_body,
        out_shape=jax.ShapeDtypeStruct((b0, b1, DIM), jnp.float32),
        grid=(b0 // _RB,),
        in_specs=[pl.BlockSpec((PADB * _RB, 128), lambda i: (i, 0))],
        out_specs=pl.BlockSpec((_RB, b1, DIM), lambda i: (i, 0, 0)),
    )(staged)
    return out


# speedup vs baseline: 3.5327x; 1.3826x over previous
import jax
import jax.numpy as jnp
from jax import lax
from jax.experimental import pallas as pl
from jax.experimental.pallas import tpu as pltpu
from jax.experimental.pallas import tpu_sc as plsc

NUM_ROWS = 1_000_000
DIM = 32
LANES = 128

_info = plsc.get_sparse_core_info()
_NC = _info.num_cores
_NS = _info.num_subcores
_NW = _NC * _NS

_JB = 4
_IBLK = 4


def _gather_body(table_hbm, xt_hbm, out_hbm, idx_v, rows_v, outt_v, sem):
    b1, b0 = xt_hbm.shape
    slab = _IBLK * LANES
    n_jc = b1 // _JB
    kt_n = DIM // 8
    wid = lax.axis_index("s") * _NC + lax.axis_index("c")
    i0 = wid * slab

    rid = [lax.iota(jnp.int32, 16) + 16 * b8 for b8 in range(8)]

    def transpose_unit(buf):
        def kstep(k, carry):
            kvec = jnp.zeros((16,), jnp.int32) + k
            for b8 in range(8):
                v = plsc.load_gather(rows_v.at[buf], [rid[b8], kvec])
                plsc.store_scatter(outt_v, [kvec, rid[b8]], v)
            return carry
        lax.fori_loop(0, DIM, kstep, 0)

    def start_gather(jj, b, buf):
        return pltpu.async_copy(
            table_hbm.at[idx_v.at[jj, pl.ds(LANES * b, LANES)]],
            rows_v.at[buf],
            sem,
        )

    def writeback(j, b):
        for kt in range(kt_n):
            r0 = (j * kt_n + kt) * (b0 // LANES) * 8 + (wid * _IBLK + b) * 8
            pltpu.sync_copy(
                outt_v.at[pl.ds(8 * kt, 8)],
                out_hbm.at[pl.ds(r0, 8)],
            )

    def jchunk(jc, carry):
        pltpu.sync_copy(xt_hbm.at[pl.ds(jc * _JB, _JB), pl.ds(i0, slab)], idx_v)
        n_u = _JB * _IBLK
        cps = [None] * n_u
        cps[0] = start_gather(0, 0, 0)
        for u in range(n_u):
            if u + 1 < n_u:
                jj, b = divmod(u + 1, _IBLK)
                cps[u + 1] = start_gather(jj, b, (u + 1) % 2)
            cps[u].wait()
            transpose_unit(u % 2)
            jj, b = divmod(u, _IBLK)
            writeback(jc * _JB + jj, b)
        return carry

    lax.fori_loop(0, n_jc, jchunk, 0)


def kernel(x, weight):
    b0, b1 = x.shape
    xt = x.T.astype(jnp.int32)
    gather = pl.kernel(
        _gather_body,
        out_type=jax.ShapeDtypeStruct((b0 * b1 * DIM // LANES, LANES),
                                      jnp.float32),
        mesh=plsc.VectorSubcoreMesh(core_axis_name="c", subcore_axis_name="s"),
        scratch_types=[
            pltpu.VMEM((_JB, _IBLK * LANES), jnp.int32),
            pltpu.VMEM((2, LANES, DIM), jnp.float32),
            pltpu.VMEM((DIM, LANES), jnp.float32),
            pltpu.SemaphoreType.DMA,
        ],
        compiler_params=pltpu.CompilerParams(use_tc_tiling_on_sc=False, needs_layout_passes=False),
    )
    out2d = gather(weight, xt)
    v5 = out2d.reshape(b1, DIM // 8, b0 // LANES, 8, LANES)
    return v5.transpose(2, 4, 0, 1, 3).reshape(b0, b1, DIM)


# speedup vs baseline: 3.8183x; 1.0809x over previous
import jax
import jax.numpy as jnp
from jax import lax
from jax.experimental import pallas as pl
from jax.experimental.pallas import tpu as pltpu
from jax.experimental.pallas import tpu_sc as plsc

NUM_ROWS = 1_000_000
DIM = 32
LANES = 128

_info = plsc.get_sparse_core_info()
_NC = _info.num_cores
_NS = _info.num_subcores
_NW = _NC * _NS

_JB = 4
_IBLK = 4


def _gather_body(table_hbm, xt_hbm, out_hbm, idx_v, rows_v, outt_v,
                 gsem, wsem0, wsem1):
    b1, b0 = xt_hbm.shape
    slab = _IBLK * LANES
    n_jc = b1 // _JB
    kt_n = DIM // 8
    wid = lax.axis_index("s") * _NC + lax.axis_index("c")
    i0 = wid * slab
    wsems = [wsem0, wsem1]

    rid = [lax.iota(jnp.int32, 16) + 16 * b8 for b8 in range(8)]

    def transpose_unit(buf):
        def kstep(k, carry):
            kvec = jnp.zeros((16,), jnp.int32) + k
            ktv = jnp.zeros((16,), jnp.int32) + k // 8
            ksv = jnp.zeros((16,), jnp.int32) + k % 8
            for b8 in range(8):
                v = plsc.load_gather(rows_v.at[buf], [rid[b8], kvec])
                plsc.store_scatter(outt_v.at[buf], [ktv, ksv, rid[b8]], v)
            return carry
        lax.fori_loop(0, DIM, kstep, 0)

    def start_gather(jj, b, buf):
        return pltpu.async_copy(
            table_hbm.at[idx_v.at[jj, pl.ds(LANES * b, LANES)]],
            rows_v.at[buf],
            gsem,
        )

    def start_writeback(j, b, buf):
        return pltpu.async_copy(
            outt_v.at[buf],
            out_hbm.at[pl.ds(j * kt_n, kt_n),
                       pl.ds((wid * _IBLK + b) * 8, 8)],
            wsems[buf],
        )

    def jchunk(jc, carry):
        pltpu.sync_copy(xt_hbm.at[pl.ds(jc * _JB, _JB), pl.ds(i0, slab)],
                        idx_v)
        n_u = _JB * _IBLK
        gcps = [None] * n_u
        wcps = [None] * n_u
        gcps[0] = start_gather(0, 0, 0)
        for u in range(n_u):
            if u + 1 < n_u:
                jj, b = divmod(u + 1, _IBLK)
                gcps[u + 1] = start_gather(jj, b, (u + 1) % 2)
            gcps[u].wait()
            if u >= 2:
                wcps[u - 2].wait()
            transpose_unit(u % 2)
            jj, b = divmod(u, _IBLK)
            wcps[u] = start_writeback(jc * _JB + jj, b, u % 2)
        wcps[n_u - 2].wait()
        wcps[n_u - 1].wait()
        return carry

    lax.fori_loop(0, n_jc, jchunk, 0)


def kernel(x, weight):
    b0, b1 = x.shape
    xt = x.T.astype(jnp.int32)
    gather = pl.kernel(
        _gather_body,
        out_type=jax.ShapeDtypeStruct((b1 * (DIM // 8), b0 // LANES * 8,
                                       LANES), jnp.float32),
        mesh=plsc.VectorSubcoreMesh(core_axis_name="c", subcore_axis_name="s"),
        scratch_types=[
            pltpu.VMEM((_JB, _IBLK * LANES), jnp.int32),
            pltpu.VMEM((2, LANES, DIM), jnp.float32),
            pltpu.VMEM((2, DIM // 8, 8, LANES), jnp.float32),
            pltpu.SemaphoreType.DMA,
            pltpu.SemaphoreType.DMA,
            pltpu.SemaphoreType.DMA,
        ],
        compiler_params=pltpu.CompilerParams(use_tc_tiling_on_sc=False,
                                             needs_layout_passes=False),
    )
    out3d = gather(weight, xt)
    v5 = out3d.reshape(b1, DIM // 8, b0 // LANES, 8, LANES)
    return v5.transpose(2, 4, 0, 1, 3).reshape(b0, b1, DIM)
